# Initial kernel scaffold; baseline (speedup 1.0000x reference)
import jax
import jax.numpy as jnp
from jax.experimental import pallas as pl


def kernel(points, rois):
    raise NotImplementedError("write your pallas kernel here")


# speedup vs baseline: 8.3744x; 8.3744x over previous
import functools

import jax
import jax.numpy as jnp
import numpy as np
from jax.experimental import pallas as pl
from jax.experimental.pallas import tpu as pltpu

_NUM_KEYPOINTS = 2048
_SAMPLE_RADIUS = 1.6
_NUM_SECTORS = 6
_N_POINTS = 100000
_N_ROIS = 128

_ROWS = 784
_PAD = _ROWS * 128
_BUF_ROWS = (_NUM_SECTORS + 1) * _NUM_KEYPOINTS // 128


def _fps_kernel(px_ref, py_ref, pz_ref, sec_ref,
                cx_ref, cy_ref, cz_ref, thr_ref,
                buf_ref, num_ref):
    f32 = jnp.float32
    i32 = jnp.int32
    shape = (_ROWS, 128)
    row_iota = jax.lax.broadcasted_iota(i32, shape, 0)
    col_iota = jax.lax.broadcasted_iota(i32, shape, 1)
    pt_iota = row_iota * 128 + col_iota
    lane1 = jax.lax.broadcasted_iota(i32, (1, 128), 1)

    x = px_ref[...]
    y = py_ref[...]
    z = pz_ref[...]
    sec = sec_ref[...]

    def roi_body(r, carry):
        cur_s, tsel = carry
        dx = x - cx_ref[r]
        dy = y - cy_ref[r]
        dz = z - cz_ref[r]
        d2 = (dx * dx + dy * dy) + dz * dz
        s = jnp.sqrt(d2)
        lt = s < cur_s
        cur_s = jnp.where(lt, s, cur_s)
        tsel = jnp.where(lt, thr_ref[r], tsel)
        return cur_s, tsel

    init = (jnp.full(shape, jnp.inf, f32), jnp.zeros(shape, f32))
    cur_s, tsel = jax.lax.fori_loop(0, _N_ROIS, roi_body, init)
    mask = cur_s < tsel

    maski = mask.astype(i32)
    anyv = jnp.sum(maski) > 0
    valid = jnp.where(anyv, maski, (pt_iota == 0).astype(i32)) > 0

    total = jnp.sum(valid.astype(i32))

    cnts = []
    for k in range(_NUM_SECTORS):
        mk = valid & (sec == k)
        cnts.append(jnp.sum(mk.astype(i32)))
    nsamps = [jnp.minimum(c, (c * _NUM_KEYPOINTS + total - 1) // total)
              for c in cnts]
    sector_num = nsamps[0]
    for k in range(1, _NUM_SECTORS):
        sector_num = sector_num + nsamps[k]
    nsamp_fb = jnp.where(sector_num == 0,
                         jnp.minimum(jnp.int32(_NUM_KEYPOINTS), total),
                         jnp.int32(0))
    nsamps.append(nsamp_fb)

    offsets = []
    off = jnp.int32(0)
    for k in range(_NUM_SECTORS + 1):
        offsets.append(off)
        off = off + nsamps[k]
    num_ref[0, 0] = off

    buf_ref[...] = jnp.zeros((_BUF_ROWS, 128), i32)

    big = jnp.int32(_PAD)

    def store_at(pos, value):
        prow = pos // 128
        pcol = pos - prow * 128
        cur = buf_ref[pl.ds(prow, 1), :]
        buf_ref[pl.ds(prow, 1), :] = jnp.where(lane1 == pcol, value, cur)

    def coords_at(idx):
        row = idx // 128
        col = idx - row * 128
        xr = px_ref[pl.ds(row, 1), :]
        yr = py_ref[pl.ds(row, 1), :]
        zr = pz_ref[pl.ds(row, 1), :]
        sel = lane1 == col
        lx = jnp.sum(jnp.where(sel, xr, 0.0))
        ly = jnp.sum(jnp.where(sel, yr, 0.0))
        lz = jnp.sum(jnp.where(sel, zr, 0.0))
        return lx, ly, lz

    for k in range(_NUM_SECTORS + 1):
        if k < _NUM_SECTORS:
            mk = valid & (sec == k)
        else:
            mk = valid
        ns_k = nsamps[k]
        off_k = offsets[k]

        @pl.when(ns_k > 0)
        def _():
            first = jnp.min(jnp.where(mk, pt_iota, big))
            store_at(off_k, first)
            dist0 = jnp.where(mk, jnp.float32(1e10), jnp.float32(-1.0))
            lx0, ly0, lz0 = coords_at(first)

            def body(i, carry):
                dist, lx, ly, lz = carry
                dx = x - lx
                dy = y - ly
                dz = z - lz
                d = (dx * dx + dy * dy) + dz * dz
                dist = jnp.minimum(dist, d)
                m = jnp.max(dist)
                last = jnp.min(jnp.where(dist == m, pt_iota, big))
                store_at(off_k + i, last)
                nlx, nly, nlz = coords_at(last)
                return dist, nlx, nly, nlz

            jax.lax.fori_loop(1, ns_k, body, (dist0, lx0, ly0, lz0))


@jax.jit
def kernel(points, rois):
    f32 = jnp.float32
    i32 = jnp.int32

    sector_size = np.pi * 2.0 / _NUM_SECTORS
    angles = jnp.arctan2(points[:, 1], points[:, 0]) + np.pi
    sector = jnp.clip(jnp.floor(angles / sector_size), 0, _NUM_SECTORS)
    sector = sector.astype(i32)

    cz_shift = rois[:, 2] + rois[:, 5] / 2.0
    half = rois[:, 3:6] / 2.0
    thr = jnp.sqrt((half[:, 0] ** 2 + half[:, 1] ** 2) + half[:, 2] ** 2) \
        + jnp.float32(_SAMPLE_RADIUS)

    pad = _PAD - _N_POINTS
    px = jnp.pad(points[:, 0], (0, pad), constant_values=1e9).reshape(_ROWS, 128)
    py = jnp.pad(points[:, 1], (0, pad), constant_values=1e9).reshape(_ROWS, 128)
    pz = jnp.pad(points[:, 2], (0, pad), constant_values=1e9).reshape(_ROWS, 128)
    sec = jnp.pad(sector, (0, pad), constant_values=_NUM_SECTORS + 1)
    sec = sec.reshape(_ROWS, 128)

    smem = pl.BlockSpec(memory_space=pltpu.SMEM)

    buf, num = pl.pallas_call(
        _fps_kernel,
        in_specs=[pl.BlockSpec((_ROWS, 128), lambda: (0, 0))] * 4 +
                 [smem] * 4,
        out_specs=[pl.BlockSpec((_BUF_ROWS, 128), lambda: (0, 0)), smem],
        out_shape=[jax.ShapeDtypeStruct((_BUF_ROWS, 128), i32),
                   jax.ShapeDtypeStruct((1, 1), i32)],
    )(px, py, pz, sec,
      rois[:, 0].astype(f32), rois[:, 1].astype(f32), cz_shift, thr)

    n = num[0, 0]
    idx = buf.reshape(-1)[jnp.arange(_NUM_KEYPOINTS, dtype=i32) % n]
    return jnp.take(points, idx, axis=0)
